# Initial kernel scaffold; baseline (speedup 1.0000x reference)
import jax
import jax.numpy as jnp
from jax.experimental import pallas as pl


def kernel(rpn_scores, rpn_reg, xyz):
    raise NotImplementedError("write your pallas kernel here")


# speedup vs baseline: 49.3116x; 49.3116x over previous
import functools

import numpy as np
import jax
import jax.numpy as jnp
from jax.experimental import pallas as pl

_LOC_SCOPE = 3.0
_LOC_BIN_SIZE = 0.5
_NUM_HEAD_BIN = 12
_NMS_POST = 512
_NMS_THRES = 0.8
_MEAN_SIZE = (1.52563191462, 1.62856739989, 3.88311640418)
_PER = 12
_PRE1, _PRE2 = 6300, 2700
_POST1, _POST2 = 358, 154
_L1 = 6400


def _decode_body(reg_ref, xyz_ref, out_ref):
    reg = reg_ref[...]
    xyz = xyz_ref[...]
    r = reg.shape[0]
    iota = jax.lax.broadcasted_iota(jnp.int32, (r, _PER), 1)

    def first_argmax(s):
        m = jnp.max(s, axis=1, keepdims=True)
        return jnp.min(jnp.where(s == m, iota, _PER), axis=1, keepdims=True)

    def gather_bin(s, b):
        return jnp.sum(jnp.where(iota == b, s, 0.0), axis=1, keepdims=True)

    x_bin = first_argmax(reg[:, 0:12])
    z_bin = first_argmax(reg[:, 12:24])
    x_res = gather_bin(reg[:, 24:36], x_bin) * _LOC_BIN_SIZE
    z_res = gather_bin(reg[:, 36:48], z_bin) * _LOC_BIN_SIZE
    half = _LOC_BIN_SIZE * 0.5
    pos_x = x_bin.astype(jnp.float32) * _LOC_BIN_SIZE + half - _LOC_SCOPE
    pos_z = z_bin.astype(jnp.float32) * _LOC_BIN_SIZE + half - _LOC_SCOPE
    pos_x = pos_x + x_res + xyz[:, 0:1]
    pos_z = pos_z + z_res + xyz[:, 2:3]

    ry_bin = first_argmax(reg[:, 49:61])
    ry_res = gather_bin(reg[:, 61:73], ry_bin)
    apc = 2.0 * np.pi / _NUM_HEAD_BIN
    twopi = jnp.float32(2.0 * np.pi)
    ry = ry_bin.astype(jnp.float32) * apc + ry_res * (apc / 2.0)
    ry = ry - jnp.floor(ry / twopi) * twopi
    ry = jnp.where(ry > np.pi, ry - twopi, ry)

    m0, m1, m2 = _MEAN_SIZE
    h = reg[:, 73:74] * m0 + m0
    w = reg[:, 74:75] * m1 + m1
    l = reg[:, 75:76] * m2 + m2
    pos_y = xyz[:, 1:2] + reg[:, 48:49] + h * 0.5

    out_ref[:, 0:1] = pos_x
    out_ref[:, 1:2] = pos_y
    out_ref[:, 2:3] = pos_z
    out_ref[:, 3:4] = h
    out_ref[:, 4:5] = w
    out_ref[:, 5:6] = l
    out_ref[:, 6:7] = ry
    out_ref[:, 7:8] = jnp.zeros((r, 1), jnp.float32)


def _decode(reg2d, xyz2d):
    rows = reg2d.shape[0]
    blk = 2048
    out = pl.pallas_call(
        _decode_body,
        grid=(rows // blk,),
        in_specs=[
            pl.BlockSpec((blk, 76), lambda i: (i, 0)),
            pl.BlockSpec((blk, 3), lambda i: (i, 0)),
        ],
        out_specs=pl.BlockSpec((blk, 8), lambda i: (i, 0)),
        out_shape=jax.ShapeDtypeStruct((rows, 8), jnp.float32),
    )(reg2d, xyz2d)
    return out[:, :7]


def _nms_body(x1_ref, y1_ref, x2_ref, y2_ref, valid_ref, keep_ref, *, max_keep):
    x1 = x1_ref[0]
    y1 = y1_ref[0]
    x2 = x2_ref[0]
    y2 = y2_ref[0]
    valid = valid_ref[0] > 0
    rows = x1.shape[0]
    lin = (jax.lax.broadcasted_iota(jnp.int32, (rows, 128), 0) * 128
           + jax.lax.broadcasted_iota(jnp.int32, (rows, 128), 1))
    area = (x2 - x1) * (y2 - y1)
    big = jnp.int32(1 << 20)
    neg = jnp.float32(-1e30)

    def body(_, carry):
        keep, supp = carry
        alive = jnp.logical_and(valid, supp == 0)
        j = jnp.min(jnp.where(alive, lin, big))
        sel = lin == j
        x1j = jnp.max(jnp.where(sel, x1, neg))
        y1j = jnp.max(jnp.where(sel, y1, neg))
        x2j = jnp.max(jnp.where(sel, x2, neg))
        y2j = jnp.max(jnp.where(sel, y2, neg))
        aj = jnp.max(jnp.where(sel, area, neg))
        iw = jnp.maximum(jnp.minimum(x2j, x2) - jnp.maximum(x1j, x1), 0.0)
        ih = jnp.maximum(jnp.minimum(y2j, y2) - jnp.maximum(y1j, y1), 0.0)
        inter = iw * ih
        iou = inter / jnp.maximum(aj + area - inter, 1e-12)
        hit = jnp.logical_and(iou > _NMS_THRES, lin > j)
        supp = jnp.maximum(supp, jnp.where(jnp.logical_or(sel, hit), 1, 0))
        keep = jnp.maximum(keep, jnp.where(sel, 1, 0))
        return keep, supp

    zeros = jnp.zeros((rows, 128), dtype=jnp.int32)
    keep, _ = jax.lax.fori_loop(0, max_keep, body, (zeros, zeros))
    keep_ref[0] = keep


def _run_nms(x1, y1, x2, y2, valid, max_keep):
    b, length = x1.shape
    rows = length // 128
    rs = lambda a: a.reshape(b, rows, 128)
    spec = pl.BlockSpec((1, rows, 128), lambda g: (g, 0, 0))
    out = pl.pallas_call(
        functools.partial(_nms_body, max_keep=max_keep),
        grid=(b,),
        in_specs=[spec] * 5,
        out_specs=spec,
        out_shape=jax.ShapeDtypeStruct((b, rows, 128), jnp.int32),
    )(rs(x1), rs(y1), rs(x2), rs(y2), rs(valid.astype(jnp.int32)))
    return out.reshape(b, length) > 0


def _bev(cp):
    x1 = cp[..., 0] - cp[..., 3] * 0.5
    y1 = cp[..., 2] - cp[..., 5] * 0.5
    x2 = cp[..., 0] + cp[..., 3] * 0.5
    y2 = cp[..., 2] + cp[..., 5] * 0.5
    return x1, y1, x2, y2


def kernel(rpn_scores, rpn_reg, xyz):
    b, n, c = rpn_reg.shape
    props = _decode(rpn_reg.reshape(-1, c), xyz.reshape(-1, 3)).reshape(b, n, 7)

    order = jnp.argsort(-rpn_scores, axis=1, stable=True)
    dist = jnp.take_along_axis(props[..., 2], order, axis=1)
    ar = jnp.arange(n, dtype=jnp.int32)
    first_mask = (dist > 0.0) & (dist <= 40.0)
    f_cnt = jnp.sum(first_mask, axis=1).astype(jnp.int32)
    pos_first = jnp.sort(jnp.where(first_mask, ar[None], n), axis=1)

    pos1 = pos_first[:, :_L1]
    valid1 = ar[None, :_L1] < jnp.minimum(f_cnt, _PRE1)[:, None]
    cand1 = jnp.take_along_axis(order, jnp.minimum(pos1, n - 1), axis=1)
    cp1 = jnp.take_along_axis(props, cand1[..., None], axis=1)
    keep1 = _run_nms(*_bev(cp1), valid1, _POST1)

    m2 = (dist > 40.0) & (dist <= 80.0)
    m2_cnt = jnp.sum(m2, axis=1).astype(jnp.int32)
    posA2 = jnp.sort(jnp.where(m2, ar[None], n), axis=1)
    posB2 = jnp.concatenate(
        [pos_first[:, _PRE1:], jnp.full((b, _PRE1), n, dtype=pos_first.dtype)],
        axis=1)
    validA2 = ar[None] < jnp.minimum(m2_cnt, _PRE2)[:, None]
    validB2 = ar[None] < (f_cnt[:, None] - _PRE1)
    use_a = (m2_cnt != 0)[:, None]
    pos2 = jnp.where(use_a, posA2, posB2)
    valid2 = jnp.where(use_a, validA2, validB2)
    cand2 = jnp.take_along_axis(order, jnp.minimum(pos2, n - 1), axis=1)
    cp2 = jnp.take_along_axis(props, cand2[..., None], axis=1)
    keep2 = _run_nms(*_bev(cp2), valid2, _POST2)

    out_idx = jnp.zeros((b, _NMS_POST), dtype=jnp.int32)
    out_valid = jnp.zeros((b, _NMS_POST), dtype=bool)
    for k in range(b):
        c1 = jnp.cumsum(keep1[k].astype(jnp.int32))
        slots1 = jnp.where(keep1[k], c1 - 1, _NMS_POST)
        out_idx = out_idx.at[k, slots1].set(cand1[k].astype(jnp.int32), mode="drop")
        out_valid = out_valid.at[k, slots1].set(True, mode="drop")
        off = c1[-1]
        c2 = jnp.cumsum(keep2[k].astype(jnp.int32))
        slots2 = jnp.where(keep2[k], off + c2 - 1, _NMS_POST)
        out_idx = out_idx.at[k, slots2].set(cand2[k].astype(jnp.int32), mode="drop")
        out_valid = out_valid.at[k, slots2].set(True, mode="drop")

    ret_bbox = jnp.where(out_valid[..., None],
                         jnp.take_along_axis(props, out_idx[..., None], axis=1),
                         jnp.float32(0.0))
    ret_scores = jnp.where(out_valid,
                           jnp.take_along_axis(rpn_scores, out_idx, axis=1),
                           jnp.float32(0.0))
    return ret_bbox, ret_scores


# speedup vs baseline: 50.0655x; 1.0153x over previous
import functools

import numpy as np
import jax
import jax.numpy as jnp
from jax.experimental import pallas as pl

_LOC_SCOPE = 3.0
_LOC_BIN_SIZE = 0.5
_NUM_HEAD_BIN = 12
_NMS_POST = 512
_NMS_THRES = 0.8
_MEAN_SIZE = (1.52563191462, 1.62856739989, 3.88311640418)
_PER = 12
_PRE1, _PRE2 = 6300, 2700
_POST1, _POST2 = 358, 154
_L1 = 6400
_L2 = 10112


def _decode_body(reg_ref, xyz_ref, out_ref):
    reg = reg_ref[...]
    xyz = xyz_ref[...]
    r = reg.shape[0]
    iota = jax.lax.broadcasted_iota(jnp.int32, (r, _PER), 1)

    def first_argmax(s):
        m = jnp.max(s, axis=1, keepdims=True)
        return jnp.min(jnp.where(s == m, iota, _PER), axis=1, keepdims=True)

    def gather_bin(s, b):
        return jnp.sum(jnp.where(iota == b, s, 0.0), axis=1, keepdims=True)

    x_bin = first_argmax(reg[:, 0:12])
    z_bin = first_argmax(reg[:, 12:24])
    x_res = gather_bin(reg[:, 24:36], x_bin) * _LOC_BIN_SIZE
    z_res = gather_bin(reg[:, 36:48], z_bin) * _LOC_BIN_SIZE
    half = _LOC_BIN_SIZE * 0.5
    pos_x = x_bin.astype(jnp.float32) * _LOC_BIN_SIZE + half - _LOC_SCOPE
    pos_z = z_bin.astype(jnp.float32) * _LOC_BIN_SIZE + half - _LOC_SCOPE
    pos_x = pos_x + x_res + xyz[:, 0:1]
    pos_z = pos_z + z_res + xyz[:, 2:3]

    ry_bin = first_argmax(reg[:, 49:61])
    ry_res = gather_bin(reg[:, 61:73], ry_bin)
    apc = 2.0 * np.pi / _NUM_HEAD_BIN
    twopi = jnp.float32(2.0 * np.pi)
    ry = ry_bin.astype(jnp.float32) * apc + ry_res * (apc / 2.0)
    ry = ry - jnp.floor(ry / twopi) * twopi
    ry = jnp.where(ry > np.pi, ry - twopi, ry)

    m0, m1, m2 = _MEAN_SIZE
    h = reg[:, 73:74] * m0 + m0
    w = reg[:, 74:75] * m1 + m1
    l = reg[:, 75:76] * m2 + m2
    pos_y = xyz[:, 1:2] + reg[:, 48:49] + h * 0.5

    out_ref[:, 0:1] = pos_x
    out_ref[:, 1:2] = pos_y
    out_ref[:, 2:3] = pos_z
    out_ref[:, 3:4] = h
    out_ref[:, 4:5] = w
    out_ref[:, 5:6] = l
    out_ref[:, 6:7] = ry
    out_ref[:, 7:8] = jnp.zeros((r, 1), jnp.float32)


def _decode(reg2d, xyz2d):
    rows = reg2d.shape[0]
    blk = 2048
    out = pl.pallas_call(
        _decode_body,
        grid=(rows // blk,),
        in_specs=[
            pl.BlockSpec((blk, 76), lambda i: (i, 0)),
            pl.BlockSpec((blk, 3), lambda i: (i, 0)),
        ],
        out_specs=pl.BlockSpec((blk, 8), lambda i: (i, 0)),
        out_shape=jax.ShapeDtypeStruct((rows, 8), jnp.float32),
    )(reg2d, xyz2d)
    return out[:, :7]


def _nms_body(x1_ref, y1_ref, x2_ref, y2_ref, valid_ref, keep_ref, *, max_keep):
    x1 = x1_ref[0]
    y1 = y1_ref[0]
    x2 = x2_ref[0]
    y2 = y2_ref[0]
    valid = valid_ref[0] > 0
    rows = x1.shape[0]
    lin = (jax.lax.broadcasted_iota(jnp.int32, (rows, 128), 0) * 128
           + jax.lax.broadcasted_iota(jnp.int32, (rows, 128), 1))
    area = (x2 - x1) * (y2 - y1)
    big = jnp.int32(1 << 20)
    neg = jnp.float32(-1e30)

    def body(_, carry):
        keep, supp = carry
        alive = jnp.logical_and(valid, supp == 0)
        j = jnp.min(jnp.where(alive, lin, big))
        sel = lin == j
        x1j = jnp.max(jnp.where(sel, x1, neg))
        y1j = jnp.max(jnp.where(sel, y1, neg))
        x2j = jnp.max(jnp.where(sel, x2, neg))
        y2j = jnp.max(jnp.where(sel, y2, neg))
        aj = jnp.max(jnp.where(sel, area, neg))
        iw = jnp.maximum(jnp.minimum(x2j, x2) - jnp.maximum(x1j, x1), 0.0)
        ih = jnp.maximum(jnp.minimum(y2j, y2) - jnp.maximum(y1j, y1), 0.0)
        inter = iw * ih
        iou = inter / jnp.maximum(aj + area - inter, 1e-12)
        hit = jnp.logical_and(iou > _NMS_THRES, lin > j)
        supp = jnp.maximum(supp, jnp.where(jnp.logical_or(sel, hit), 1, 0))
        keep = jnp.maximum(keep, jnp.where(sel, 1, 0))
        return keep, supp

    zeros = jnp.zeros((rows, 128), dtype=jnp.int32)
    keep, _ = jax.lax.fori_loop(0, max_keep, body, (zeros, zeros))
    keep_ref[0] = keep


def _run_nms(x1, y1, x2, y2, valid, max_keep):
    b, length = x1.shape
    rows = length // 128
    rs = lambda a: a.reshape(b, rows, 128)
    spec = pl.BlockSpec((1, rows, 128), lambda g: (g, 0, 0))
    out = pl.pallas_call(
        functools.partial(_nms_body, max_keep=max_keep),
        grid=(b,),
        in_specs=[spec] * 5,
        out_specs=spec,
        out_shape=jax.ShapeDtypeStruct((b, rows, 128), jnp.int32),
    )(rs(x1), rs(y1), rs(x2), rs(y2), rs(valid.astype(jnp.int32)))
    return out.reshape(b, length) > 0


def _bev(cp):
    x1 = cp[..., 0] - cp[..., 3] * 0.5
    y1 = cp[..., 2] - cp[..., 5] * 0.5
    x2 = cp[..., 0] + cp[..., 3] * 0.5
    y2 = cp[..., 2] + cp[..., 5] * 0.5
    return x1, y1, x2, y2


def kernel(rpn_scores, rpn_reg, xyz):
    b, n, c = rpn_reg.shape
    props = _decode(rpn_reg.reshape(-1, c), xyz.reshape(-1, 3)).reshape(b, n, 7)

    order = jnp.argsort(-rpn_scores, axis=1, stable=True)
    dist = jnp.take_along_axis(props[..., 2], order, axis=1)
    ar = jnp.arange(n, dtype=jnp.int32)
    brow = jnp.arange(b, dtype=jnp.int32)[:, None]

    def compact(mask):
        slot = jnp.cumsum(mask.astype(jnp.int32), axis=1) - 1
        slot = jnp.where(mask, slot, n)
        out = jnp.full((b, n), n, dtype=jnp.int32)
        return out.at[jnp.broadcast_to(brow, (b, n)), slot].set(
            jnp.broadcast_to(ar[None], (b, n)), mode="drop")

    first_mask = (dist > 0.0) & (dist <= 40.0)
    f_cnt = jnp.sum(first_mask, axis=1).astype(jnp.int32)
    pos_first = compact(first_mask)

    pos1 = pos_first[:, :_L1]
    valid1 = ar[None, :_L1] < jnp.minimum(f_cnt, _PRE1)[:, None]
    cand1 = jnp.take_along_axis(order, jnp.minimum(pos1, n - 1), axis=1)
    cp1 = jnp.take_along_axis(props, cand1[..., None], axis=1)
    keep1 = _run_nms(*_bev(cp1), valid1, _POST1)

    m2 = (dist > 40.0) & (dist <= 80.0)
    m2_cnt = jnp.sum(m2, axis=1).astype(jnp.int32)
    posA2 = compact(m2)[:, :_L2]
    posB2 = jnp.concatenate(
        [pos_first[:, _PRE1:],
         jnp.full((b, _L2 - (n - _PRE1)), n, dtype=pos_first.dtype)],
        axis=1)
    validA2 = ar[None, :_L2] < jnp.minimum(m2_cnt, _PRE2)[:, None]
    validB2 = ar[None, :_L2] < (f_cnt[:, None] - _PRE1)
    use_a = (m2_cnt != 0)[:, None]
    pos2 = jnp.where(use_a, posA2, posB2)
    valid2 = jnp.where(use_a, validA2, validB2)
    cand2 = jnp.take_along_axis(order, jnp.minimum(pos2, n - 1), axis=1)
    cp2 = jnp.take_along_axis(props, cand2[..., None], axis=1)
    keep2 = _run_nms(*_bev(cp2), valid2, _POST2)

    out_idx = jnp.zeros((b, _NMS_POST), dtype=jnp.int32)
    out_valid = jnp.zeros((b, _NMS_POST), dtype=bool)
    for k in range(b):
        c1 = jnp.cumsum(keep1[k].astype(jnp.int32))
        slots1 = jnp.where(keep1[k], c1 - 1, _NMS_POST)
        out_idx = out_idx.at[k, slots1].set(cand1[k].astype(jnp.int32), mode="drop")
        out_valid = out_valid.at[k, slots1].set(True, mode="drop")
        off = c1[-1]
        c2 = jnp.cumsum(keep2[k].astype(jnp.int32))
        slots2 = jnp.where(keep2[k], off + c2 - 1, _NMS_POST)
        out_idx = out_idx.at[k, slots2].set(cand2[k].astype(jnp.int32), mode="drop")
        out_valid = out_valid.at[k, slots2].set(True, mode="drop")

    ret_bbox = jnp.where(out_valid[..., None],
                         jnp.take_along_axis(props, out_idx[..., None], axis=1),
                         jnp.float32(0.0))
    ret_scores = jnp.where(out_valid,
                           jnp.take_along_axis(rpn_scores, out_idx, axis=1),
                           jnp.float32(0.0))
    return ret_bbox, ret_scores
